# baseline (speedup 1.0000x reference)
import functools

import jax
import jax.numpy as jnp
from jax import lax
from jax.experimental import pallas as pl
from jax.experimental.pallas import tpu as pltpu
from jax.experimental.pallas import tpu_sc as plsc

N = 10000
E = 320000
NP = 10240
CH = 128
EP = 327680
EROWS = EP // CH
NC = 2
NS = 16
RPT = NP // NS

_F32 = jnp.float32


def _zero_vmem_2d(ref, nrows):
    def row(i, _):
        for g in range(8):
            ref[i, pl.ds(g * 16, 16)] = jnp.zeros((16,), _F32)
        return 0
    lax.fori_loop(0, nrows, row, 0)


def _make_sc_agg(rows_per_tile, edge_split, compute_deg, table_rows,
                 core_table_offset):
    mesh = plsc.VectorSubcoreMesh(core_axis_name="c", subcore_axis_name="s")

    out_type = [jax.ShapeDtypeStruct((NC, NP, 128), _F32)]
    if compute_deg:
        out_type.append(jax.ShapeDtypeStruct((NC, NP), _F32))

    W = 8 if compute_deg else 16
    scratch = [
        pltpu.VMEM((W, 2, CH), jnp.int32),
        pltpu.VMEM((W, 2, CH), jnp.int32),
        pltpu.VMEM((CH, 128), _F32),
        pltpu.VMEM((CH, 128), _F32),
    ]
    if compute_deg:
        scratch.append(pltpu.VMEM((CH,), _F32))
        scratch.append(pltpu.VMEM((RPT,), _F32))
        scratch.append(pltpu.VMEM_SHARED((NP,), _F32))
    scratch.append(pltpu.VMEM_SHARED((NP, 128), _F32))
    scratch += [pltpu.SemaphoreType.DMA] * 6

    @functools.partial(pl.kernel, mesh=mesh, out_type=tuple(out_type),
                       scratch_types=scratch)
    def sc_agg(table, idx3d, raw_out, *rest):
        if compute_deg:
            (deg_out, idxA, idxB, rb0, rb1, ones_v, zero_v, dacc, acc,
             gs0, gs1, ssem, dsem, isA, isB) = rest
        else:
            deg_out = ones_v = zero_v = dacc = None
            idxA, idxB, rb0, rb1, acc, gs0, gs1, ssem, dsem, isA, isB = rest
        c = lax.axis_index("c")
        s = lax.axis_index("s")

        _zero_vmem_2d(rb0, CH)
        if compute_deg:
            for i in range(CH // 16):
                ones_v[pl.ds(i * 16, 16)] = jnp.ones((16,), _F32)
            for i in range(RPT // 16):
                zero_v[pl.ds(i * 16, 16)] = jnp.zeros((16,), _F32)
            pltpu.sync_copy(zero_v, dacc.at[pl.ds(s * RPT, RPT)])

        for k in range(RPT // CH):
            pltpu.sync_copy(rb0, acc.at[pl.ds(s * RPT + k * CH, CH)])

        plsc.subcore_barrier()

        w = c * NS + s if edge_split else s
        base = w * rows_per_tile
        off = c * core_table_offset
        bufs = (rb0, rb1)
        gsems = (gs0, gs1)
        nwin = rows_per_tile // W

        def islice(j):
            return idx3d.at[pl.ds(pl.multiple_of(base + j * W, 8), W)]

        def do_window(idx_w):
            if core_table_offset:
                for i in range(W):
                    for g in range(8):
                        sl = pl.ds(g * 16, 16)
                        idx_w[i, 0, sl] = idx_w[i, 0, sl] + off
            gd = [None, None]
            sd = [None, None]
            degd = []
            gd[0] = pltpu.async_copy(table.at[idx_w.at[0, 0]], rb0, gs0)
            for k in range(W):
                b = k & 1
                nb = (k + 1) & 1
                if k + 1 < W:
                    if sd[nb] is not None:
                        sd[nb].wait()
                        sd[nb] = None
                    gd[nb] = pltpu.async_copy(
                        table.at[idx_w.at[k + 1, 0]], bufs[nb], gsems[nb])
                gd[b].wait()
                sd[b] = pltpu.async_copy(bufs[b], acc.at[idx_w.at[k, 1]],
                                         ssem, add=True)
                if compute_deg:
                    degd.append(pltpu.async_copy(
                        ones_v, dacc.at[idx_w.at[k, 1]], dsem, add=True))
            for b in range(2):
                if sd[b] is not None:
                    sd[b].wait()
            for d in degd:
                d.wait()

        pltpu.async_copy(islice(0), idxA, isA)

        def pair(jj, _):
            j0 = jj * 2
            pltpu.make_async_copy(islice(j0), idxA, isA).wait()
            pltpu.async_copy(islice(j0 + 1), idxB, isB)
            do_window(idxA)
            pltpu.make_async_copy(islice(j0 + 1), idxB, isB).wait()

            @pl.when(j0 + 2 < nwin)
            def _prefetch():
                pltpu.async_copy(islice(j0 + 2), idxA, isA)
            do_window(idxB)
            return 0
        lax.fori_loop(0, nwin // 2, pair, 0)
        if nwin % 2:
            pltpu.make_async_copy(islice(nwin - 1), idxA, isA).wait()
            do_window(idxA)

        plsc.subcore_barrier()

        pltpu.sync_copy(acc.at[pl.ds(s * RPT, RPT)],
                        raw_out.at[c, pl.ds(s * RPT, RPT)])
        if compute_deg:
            pltpu.sync_copy(dacc.at[pl.ds(s * RPT, RPT)],
                            deg_out.at[c].at[pl.ds(s * RPT, RPT)])

    return sc_agg


def _dgT(a, w):
    return lax.dot_general(a, w, (((1,), (1,)), ((), ())),
                           preferred_element_type=_F32)


def _bn_relu(h, g, be):
    hv = h[:N]
    mean = jnp.mean(hv, axis=0, keepdims=True)
    cent = hv - mean
    var = jnp.mean(cent * cent, axis=0, keepdims=True)
    hn = (h - mean) * lax.rsqrt(var + 1e-5) * g + be
    return jnp.maximum(hn, 0.0)


def _dense1_body(raw_ref, x_ref, r_ref, wl_ref, bl_ref, wr_ref, g_ref,
                 be_ref, out_ref):
    agg = (raw_ref[0] + raw_ref[1]) * r_ref[...]
    h = _dgT(agg, wl_ref[...]) + bl_ref[...] + _dgT(x_ref[...], wr_ref[...])
    h1 = _bn_relu(h, g_ref[...], be_ref[...])
    out_ref[0] = h1[:, :128]
    out_ref[1] = h1[:, 128:]


def _dense2_body(raw_ref, h1_ref, r_ref, wl2_ref, bl2_ref, wr2_ref, g_ref,
                 be_ref, wl3_ref, t_ref, h2_ref):
    agg = jnp.concatenate([raw_ref[0], raw_ref[1]], axis=1) * r_ref[...]
    h1f = jnp.concatenate([h1_ref[0], h1_ref[1]], axis=1)
    h = _dgT(agg, wl2_ref[...]) + bl2_ref[...] + _dgT(h1f, wr2_ref[...])
    h2 = _bn_relu(h, g_ref[...], be_ref[...])
    t_ref[...] = _dgT(h2, wl3_ref[...])
    h2_ref[...] = h2


def _dense_r3_body(h2_ref, wr3_ref, bl3_ref, r3_ref):
    r3_ref[...] = _dgT(h2_ref[...], wr3_ref[...]) + bl3_ref[...]


def _dense3_body(raw_ref, r_ref, r3_ref, out_ref):
    out_ref[...] = ((raw_ref[0][:N] + raw_ref[1][:N]) * r_ref[:N]
                    + r3_ref[:N])


def kernel(x, edge_index, W_l1, b_l1, W_r1, g1, be1, W_l2, b_l2, W_r2,
           g2, be2, W_l3, b_l3, W_r3):
    ei = edge_index.astype(jnp.int32)
    pad = N + jnp.arange(EP - E, dtype=jnp.int32) % (NP - N)
    src2d = jnp.concatenate([ei[0], pad]).reshape(EROWS, CH)
    dst2d = jnp.concatenate([ei[1], pad]).reshape(EROWS, CH)
    idx3d = jnp.stack([src2d, dst2d], axis=1)
    xp = jnp.zeros((NP, 128), _F32).at[:N].set(x)

    sc1 = _make_sc_agg(EROWS // (NC * NS), edge_split=True, compute_deg=True,
                       table_rows=NP, core_table_offset=0)
    raw1, degp = sc1(xp, idx3d)
    deg = degp[0] + degp[1]
    recip = (1.0 / jnp.maximum(deg, 1.0)).reshape(NP, 1)

    h1 = pl.pallas_call(
        _dense1_body,
        out_shape=jax.ShapeDtypeStruct((NC, NP, 128), _F32),
    )(raw1, xp, recip, W_l1, b_l1.reshape(1, -1), W_r1,
      g1.reshape(1, -1), be1.reshape(1, -1))

    sc2 = _make_sc_agg(EROWS // NS, edge_split=False, compute_deg=False,
                       table_rows=NC * NP, core_table_offset=NP)
    (raw2,) = sc2(h1.reshape(NC * NP, 128), idx3d)

    t, h2 = pl.pallas_call(
        _dense2_body,
        out_shape=(jax.ShapeDtypeStruct((NP, 128), _F32),
                   jax.ShapeDtypeStruct((NP, 256), _F32)),
    )(raw2, h1, recip, W_l2, b_l2.reshape(1, -1), W_r2,
      g2.reshape(1, -1), be2.reshape(1, -1), W_l3)

    sc3 = _make_sc_agg(EROWS // (NC * NS), edge_split=True,
                       compute_deg=False, table_rows=NP, core_table_offset=0)
    (raw3,) = sc3(t, idx3d)

    r3 = pl.pallas_call(
        _dense_r3_body,
        out_shape=jax.ShapeDtypeStruct((NP, 128), _F32),
    )(h2, W_r3, b_l3.reshape(1, -1))

    out = pl.pallas_call(
        _dense3_body,
        out_shape=jax.ShapeDtypeStruct((N, 128), _F32),
    )(raw3, recip, r3)
    return out


# speedup vs baseline: 1.0098x; 1.0098x over previous
import functools

import jax
import jax.numpy as jnp
from jax import lax
from jax.experimental import pallas as pl
from jax.experimental.pallas import tpu as pltpu
from jax.experimental.pallas import tpu_sc as plsc

N = 10000
E = 320000
NP = 10240
CH = 128
EP = 327680
EROWS = EP // CH
NC = 2
NS = 16
RPT = NP // NS

_F32 = jnp.float32


def _zero_vmem_2d(ref, nrows):
    def row(i, _):
        for g in range(8):
            ref[i, pl.ds(g * 16, 16)] = jnp.zeros((16,), _F32)
        return 0
    lax.fori_loop(0, nrows, row, 0)


def _make_sc_agg(rows_per_tile, edge_split, compute_deg, table_rows,
                 core_table_offset):
    mesh = plsc.VectorSubcoreMesh(core_axis_name="c", subcore_axis_name="s")

    out_type = [jax.ShapeDtypeStruct((NC, NP, 128), _F32)]
    if compute_deg:
        out_type.append(jax.ShapeDtypeStruct((NC, NP), _F32))

    W = 16
    scratch = [
        pltpu.VMEM((W, 2, CH), jnp.int32),
        pltpu.VMEM((W, 2, CH), jnp.int32),
        pltpu.VMEM((CH, 128), _F32),
        pltpu.VMEM((CH, 128), _F32),
    ]
    if compute_deg:
        scratch.append(pltpu.VMEM((CH,), _F32))
        scratch.append(pltpu.VMEM((RPT,), _F32))
        scratch.append(pltpu.VMEM_SHARED((NP,), _F32))
    scratch.append(pltpu.VMEM_SHARED((NP, 128), _F32))
    scratch += [pltpu.SemaphoreType.DMA] * 6

    @functools.partial(pl.kernel, mesh=mesh, out_type=tuple(out_type),
                       scratch_types=scratch)
    def sc_agg(table, idx3d, raw_out, *rest):
        if compute_deg:
            (deg_out, idxA, idxB, rb0, rb1, ones_v, zero_v, dacc, acc,
             gs0, gs1, ssem, dsem, isA, isB) = rest
        else:
            deg_out = ones_v = zero_v = dacc = None
            idxA, idxB, rb0, rb1, acc, gs0, gs1, ssem, dsem, isA, isB = rest
        c = lax.axis_index("c")
        s = lax.axis_index("s")

        _zero_vmem_2d(rb0, CH)
        if compute_deg:
            for i in range(CH // 16):
                ones_v[pl.ds(i * 16, 16)] = jnp.ones((16,), _F32)
            for i in range(RPT // 16):
                zero_v[pl.ds(i * 16, 16)] = jnp.zeros((16,), _F32)
            pltpu.sync_copy(zero_v, dacc.at[pl.ds(s * RPT, RPT)])

        for k in range(RPT // CH):
            pltpu.sync_copy(rb0, acc.at[pl.ds(s * RPT + k * CH, CH)])

        plsc.subcore_barrier()

        w = c * NS + s if edge_split else s
        base = w * rows_per_tile
        off = c * core_table_offset
        bufs = (rb0, rb1)
        gsems = (gs0, gs1)
        nwin = rows_per_tile // W

        def islice(j):
            return idx3d.at[pl.ds(pl.multiple_of(base + j * W, 8), W)]

        def do_window(idx_w):
            if core_table_offset:
                for i in range(W):
                    for g in range(8):
                        sl = pl.ds(g * 16, 16)
                        idx_w[i, 0, sl] = idx_w[i, 0, sl] + off
            gd = [None, None]
            sd = [None, None]
            degd = []
            gd[0] = pltpu.async_copy(table.at[idx_w.at[0, 0]], rb0, gs0)
            for k in range(W):
                b = k & 1
                nb = (k + 1) & 1
                if k + 1 < W:
                    if sd[nb] is not None:
                        sd[nb].wait()
                        sd[nb] = None
                    gd[nb] = pltpu.async_copy(
                        table.at[idx_w.at[k + 1, 0]], bufs[nb], gsems[nb])
                gd[b].wait()
                sd[b] = pltpu.async_copy(bufs[b], acc.at[idx_w.at[k, 1]],
                                         ssem, add=True)
                if compute_deg:
                    degd.append(pltpu.async_copy(
                        ones_v, dacc.at[idx_w.at[k, 1]], dsem, add=True))
            for b in range(2):
                if sd[b] is not None:
                    sd[b].wait()
            for d in degd:
                d.wait()

        pltpu.async_copy(islice(0), idxA, isA)

        def pair(jj, _):
            j0 = jj * 2
            pltpu.make_async_copy(islice(j0), idxA, isA).wait()
            pltpu.async_copy(islice(j0 + 1), idxB, isB)
            do_window(idxA)
            pltpu.make_async_copy(islice(j0 + 1), idxB, isB).wait()

            @pl.when(j0 + 2 < nwin)
            def _prefetch():
                pltpu.async_copy(islice(j0 + 2), idxA, isA)
            do_window(idxB)
            return 0
        lax.fori_loop(0, nwin // 2, pair, 0)
        if nwin % 2:
            pltpu.make_async_copy(islice(nwin - 1), idxA, isA).wait()
            do_window(idxA)

        plsc.subcore_barrier()

        pltpu.sync_copy(acc.at[pl.ds(s * RPT, RPT)],
                        raw_out.at[c, pl.ds(s * RPT, RPT)])
        if compute_deg:
            pltpu.sync_copy(dacc.at[pl.ds(s * RPT, RPT)],
                            deg_out.at[c].at[pl.ds(s * RPT, RPT)])

    return sc_agg


def _dgT(a, w):
    return lax.dot_general(a, w, (((1,), (1,)), ((), ())),
                           preferred_element_type=_F32)


def _bn_relu(h, g, be):
    hv = h[:N]
    mean = jnp.mean(hv, axis=0, keepdims=True)
    cent = hv - mean
    var = jnp.mean(cent * cent, axis=0, keepdims=True)
    hn = (h - mean) * lax.rsqrt(var + 1e-5) * g + be
    return jnp.maximum(hn, 0.0)


def _dense1_body(raw_ref, x_ref, r_ref, wl_ref, bl_ref, wr_ref, g_ref,
                 be_ref, out_ref):
    agg = (raw_ref[0] + raw_ref[1]) * r_ref[...]
    h = _dgT(agg, wl_ref[...]) + bl_ref[...] + _dgT(x_ref[...], wr_ref[...])
    h1 = _bn_relu(h, g_ref[...], be_ref[...])
    out_ref[0] = h1[:, :128]
    out_ref[1] = h1[:, 128:]


def _dense2_body(raw_ref, h1_ref, r_ref, wl2_ref, bl2_ref, wr2_ref, g_ref,
                 be_ref, wl3_ref, t_ref, h2_ref):
    agg = jnp.concatenate([raw_ref[0], raw_ref[1]], axis=1) * r_ref[...]
    h1f = jnp.concatenate([h1_ref[0], h1_ref[1]], axis=1)
    h = _dgT(agg, wl2_ref[...]) + bl2_ref[...] + _dgT(h1f, wr2_ref[...])
    h2 = _bn_relu(h, g_ref[...], be_ref[...])
    t_ref[...] = _dgT(h2, wl3_ref[...])
    h2_ref[...] = h2


def _dense_r3_body(h2_ref, wr3_ref, bl3_ref, r3_ref):
    r3_ref[...] = _dgT(h2_ref[...], wr3_ref[...]) + bl3_ref[...]


def _dense3_body(raw_ref, r_ref, r3_ref, out_ref):
    out_ref[...] = ((raw_ref[0][:N] + raw_ref[1][:N]) * r_ref[:N]
                    + r3_ref[:N])


def kernel(x, edge_index, W_l1, b_l1, W_r1, g1, be1, W_l2, b_l2, W_r2,
           g2, be2, W_l3, b_l3, W_r3):
    ei = edge_index.astype(jnp.int32)
    pad = N + jnp.arange(EP - E, dtype=jnp.int32) % (NP - N)
    src2d = jnp.concatenate([ei[0], pad]).reshape(EROWS, CH)
    dst2d = jnp.concatenate([ei[1], pad]).reshape(EROWS, CH)
    idx3d = jnp.stack([src2d, dst2d], axis=1)
    xp = jnp.zeros((NP, 128), _F32).at[:N].set(x)

    sc1 = _make_sc_agg(EROWS // (NC * NS), edge_split=True, compute_deg=True,
                       table_rows=NP, core_table_offset=0)
    raw1, degp = sc1(xp, idx3d)
    deg = degp[0] + degp[1]
    recip = (1.0 / jnp.maximum(deg, 1.0)).reshape(NP, 1)

    h1 = pl.pallas_call(
        _dense1_body,
        out_shape=jax.ShapeDtypeStruct((NC, NP, 128), _F32),
    )(raw1, xp, recip, W_l1, b_l1.reshape(1, -1), W_r1,
      g1.reshape(1, -1), be1.reshape(1, -1))

    sc2 = _make_sc_agg(EROWS // NS, edge_split=False, compute_deg=False,
                       table_rows=NC * NP, core_table_offset=NP)
    (raw2,) = sc2(h1.reshape(NC * NP, 128), idx3d)

    t, h2 = pl.pallas_call(
        _dense2_body,
        out_shape=(jax.ShapeDtypeStruct((NP, 128), _F32),
                   jax.ShapeDtypeStruct((NP, 256), _F32)),
    )(raw2, h1, recip, W_l2, b_l2.reshape(1, -1), W_r2,
      g2.reshape(1, -1), be2.reshape(1, -1), W_l3)

    sc3 = _make_sc_agg(EROWS // (NC * NS), edge_split=True,
                       compute_deg=False, table_rows=NP, core_table_offset=0)
    (raw3,) = sc3(t, idx3d)

    r3 = pl.pallas_call(
        _dense_r3_body,
        out_shape=jax.ShapeDtypeStruct((NP, 128), _F32),
    )(h2, W_r3, b_l3.reshape(1, -1))

    out = pl.pallas_call(
        _dense3_body,
        out_shape=jax.ShapeDtypeStruct((N, 128), _F32),
    )(raw3, recip, r3)
    return out
